# Initial kernel scaffold; baseline (speedup 1.0000x reference)
import jax
import jax.numpy as jnp
from jax.experimental import pallas as pl


def kernel(dense, emb_idx, tables, W1, b1, W2, b2, T1, tb1, T2, tb2, T3, tb3):
    raise NotImplementedError("write your pallas kernel here")


# speedup vs baseline: 21.5038x; 21.5038x over previous
import functools

import jax
import jax.numpy as jnp
from jax import lax
from jax.experimental import pallas as pl
from jax.experimental.pallas import tpu as pltpu
from jax.experimental.pallas import tpu_sc as plsc

_EMB_DIM = 128
_NUM_DENSE = 13
_NUM_TABLES = 26
_VOCAB = 10000
_B = 16384

_NC = 2
_NS = 16
_NW = _NC * _NS

_CH = 128
_N_ROWS = _NUM_TABLES * _B
_ROWS_PER_W = _N_ROWS // _NW
_CH_PER_W = _ROWS_PER_W // _CH
_CH_PER_TABLE = _B // _CH


def _sc_gather(tables_flat, idx_flat):
    mesh = plsc.VectorSubcoreMesh(core_axis_name="c", subcore_axis_name="s")

    @functools.partial(
        pl.kernel,
        mesh=mesh,
        out_type=jax.ShapeDtypeStruct((_N_ROWS, _EMB_DIM), jnp.float32),
        scratch_types=[
            pltpu.VMEM((2, _CH), jnp.int32),
            pltpu.VMEM((2, _CH, _EMB_DIM), jnp.float32),
            pltpu.SemaphoreType.DMA,
            pltpu.SemaphoreType.DMA,
        ],
    )
    def gather(tab_hbm, idx_hbm, out_hbm, idx_v, rows_v, gsem0, gsem1):
        wid = lax.axis_index("s") * _NC + lax.axis_index("c")
        g0 = wid * _CH_PER_W
        gsems = (gsem0, gsem1)

        def load_idx(g, buf):
            pltpu.sync_copy(idx_hbm.at[pl.ds(g * _CH, _CH)], idx_v.at[buf])
            off = (g // _CH_PER_TABLE) * _VOCAB
            for j in range(_CH // 16):
                sl = pl.ds(j * 16, 16)
                idx_v[buf, sl] = idx_v[buf, sl] + off

        def fire(g, buf):
            pltpu.async_copy(tab_hbm.at[idx_v.at[buf]], rows_v.at[buf],
                             gsems[buf])

        def wait_and_store(g, buf):
            pltpu.make_async_copy(tab_hbm.at[idx_v.at[buf]], rows_v.at[buf],
                                  gsems[buf]).wait()
            pltpu.sync_copy(rows_v.at[buf], out_hbm.at[pl.ds(g * _CH, _CH)])

        load_idx(g0, 0)
        fire(g0, 0)

        def body(p, carry):
            i = 2 * p
            g = g0 + i
            load_idx(g + 1, 1)
            fire(g + 1, 1)
            wait_and_store(g, 0)

            @pl.when(p < _CH_PER_W // 2 - 1)
            def _():
                load_idx(g + 2, 0)
                fire(g + 2, 0)

            wait_and_store(g + 1, 1)
            return carry

        lax.fori_loop(0, _CH_PER_W // 2, body, 0)

    return gather(tables_flat, idx_flat)


_BT = 512


def _mlp_body(dense_r, emb_r, W1_r, b1_r, W2_r, b2_r, T1_r, tb1_r, T2_r,
              tb2_r, T3_r, tb3_r, out_r):
    f32 = jnp.float32
    h = jnp.maximum(
        jnp.dot(dense_r[...], W1_r[...], preferred_element_type=f32) + b1_r[...],
        0.0)
    bot = jnp.dot(h, W2_r[...], preferred_element_type=f32) + b2_r[...]
    acc = jnp.dot(bot, T1_r[0:_EMB_DIM, :], preferred_element_type=f32)
    for t in range(_NUM_TABLES):
        acc = acc + jnp.dot(
            emb_r[t],
            T1_r[(t + 1) * _EMB_DIM:(t + 2) * _EMB_DIM, :],
            preferred_element_type=f32)
    h1 = jnp.maximum(acc + tb1_r[...], 0.0)
    h2 = jnp.maximum(
        jnp.dot(h1, T2_r[...], preferred_element_type=f32) + tb2_r[...], 0.0)
    out_r[...] = jnp.dot(h2, T3_r[...], preferred_element_type=f32) + tb3_r[...]


def _tc_mlp(dense, emb3, W1, b1, W2, b2, T1, tb1, T2, tb2, T3, tb3):
    in_top = _EMB_DIM * (_NUM_TABLES + 1)
    const = lambda i: (0, 0)
    return pl.pallas_call(
        _mlp_body,
        grid=(_B // _BT,),
        in_specs=[
            pl.BlockSpec((_BT, _NUM_DENSE), lambda i: (i, 0)),
            pl.BlockSpec((_NUM_TABLES, _BT, _EMB_DIM), lambda i: (0, i, 0)),
            pl.BlockSpec((_NUM_DENSE, _EMB_DIM), const),
            pl.BlockSpec((1, _EMB_DIM), const),
            pl.BlockSpec((_EMB_DIM, _EMB_DIM), const),
            pl.BlockSpec((1, _EMB_DIM), const),
            pl.BlockSpec((in_top, 512), const),
            pl.BlockSpec((1, 512), const),
            pl.BlockSpec((512, 256), const),
            pl.BlockSpec((1, 256), const),
            pl.BlockSpec((256, 1), const),
            pl.BlockSpec((1, 1), const),
        ],
        out_specs=pl.BlockSpec((_BT, 1), lambda i: (i, 0)),
        out_shape=jax.ShapeDtypeStruct((_B, 1), jnp.float32),
    )(dense, emb3, W1, b1.reshape(1, -1), W2, b2.reshape(1, -1), T1,
      tb1.reshape(1, -1), T2, tb2.reshape(1, -1), T3, tb3.reshape(1, -1))


def kernel(dense, emb_idx, tables, W1, b1, W2, b2, T1, tb1, T2, tb2, T3, tb3):
    tables_flat = tables.reshape(_NUM_TABLES * _VOCAB, _EMB_DIM)
    idx_flat = emb_idx.astype(jnp.int32).reshape(-1)
    emb_flat = _sc_gather(tables_flat, idx_flat)
    emb3 = emb_flat.reshape(_NUM_TABLES, _B, _EMB_DIM)
    return _tc_mlp(dense, emb3, W1, b1, W2, b2, T1, tb1, T2, tb2, T3, tb3)


# speedup vs baseline: 24.8531x; 1.1558x over previous
import functools

import jax
import jax.numpy as jnp
from jax import lax
from jax.experimental import pallas as pl
from jax.experimental.pallas import tpu as pltpu
from jax.experimental.pallas import tpu_sc as plsc

_EMB_DIM = 128
_NUM_DENSE = 13
_NUM_TABLES = 26
_VOCAB = 10000
_B = 16384

_NC = 2
_NS = 16
_NW = _NC * _NS

_CH = 128
_N_ROWS = _NUM_TABLES * _B
_ROWS_PER_W = _N_ROWS // _NW
_CH_PER_W = _ROWS_PER_W // _CH
_CH_PER_TABLE = _B // _CH


def _sc_gather(tables_flat, idx_flat):
    mesh = plsc.VectorSubcoreMesh(core_axis_name="c", subcore_axis_name="s")

    @functools.partial(
        pl.kernel,
        mesh=mesh,
        out_type=jax.ShapeDtypeStruct((_B, _NUM_TABLES * _EMB_DIM),
                                      jnp.float32),
        scratch_types=[
            pltpu.VMEM((2, _CH), jnp.int32),
            pltpu.VMEM((2, _CH, _EMB_DIM), jnp.float32),
            pltpu.SemaphoreType.DMA,
            pltpu.SemaphoreType.DMA,
        ],
    )
    def gather(tab_hbm, idx_hbm, out_hbm, idx_v, rows_v, gsem0, gsem1):
        wid = lax.axis_index("s") * _NC + lax.axis_index("c")
        g0 = wid * _CH_PER_W
        gsems = (gsem0, gsem1)

        def load_idx(g, buf):
            pltpu.sync_copy(idx_hbm.at[pl.ds(g * _CH, _CH)], idx_v.at[buf])
            off = (g // _CH_PER_TABLE) * _VOCAB
            for j in range(_CH // 16):
                sl = pl.ds(j * 16, 16)
                idx_v[buf, sl] = idx_v[buf, sl] + off

        def fire(g, buf):
            pltpu.async_copy(tab_hbm.at[idx_v.at[buf]], rows_v.at[buf],
                             gsems[buf])

        def wait_and_store(g, buf):
            pltpu.make_async_copy(tab_hbm.at[idx_v.at[buf]], rows_v.at[buf],
                                  gsems[buf]).wait()
            t = g // _CH_PER_TABLE
            c = g - t * _CH_PER_TABLE
            pltpu.sync_copy(
                rows_v.at[buf],
                out_hbm.at[pl.ds(c * _CH, _CH), pl.ds(t * _EMB_DIM, _EMB_DIM)])

        load_idx(g0, 0)
        fire(g0, 0)

        def body(p, carry):
            i = 2 * p
            g = g0 + i
            load_idx(g + 1, 1)
            fire(g + 1, 1)
            wait_and_store(g, 0)

            @pl.when(p < _CH_PER_W // 2 - 1)
            def _():
                load_idx(g + 2, 0)
                fire(g + 2, 0)

            wait_and_store(g + 1, 1)
            return carry

        lax.fori_loop(0, _CH_PER_W // 2, body, 0)

    return gather(tables_flat, idx_flat)


_BT = 512


def _mlp_body(dense_r, emb_r, W1_r, b1_r, W2_r, b2_r, T1_r, tb1_r, T2_r,
              tb2_r, T3_r, tb3_r, out_r):
    f32 = jnp.float32
    bf16 = jnp.bfloat16
    h = jnp.maximum(
        jnp.dot(dense_r[...], W1_r[...], preferred_element_type=f32) + b1_r[...],
        0.0)
    bot = jnp.dot(h, W2_r[...], preferred_element_type=f32) + b2_r[...]
    acc = jnp.dot(bot.astype(bf16), T1_r[0:_EMB_DIM, :].astype(bf16),
                  preferred_element_type=f32)
    acc = acc + jnp.dot(emb_r[...].astype(bf16),
                        T1_r[_EMB_DIM:, :].astype(bf16),
                        preferred_element_type=f32)
    h1 = jnp.maximum(acc + tb1_r[...], 0.0)
    h2 = jnp.maximum(
        jnp.dot(h1, T2_r[...], preferred_element_type=f32) + tb2_r[...], 0.0)
    out_r[...] = jnp.dot(h2, T3_r[...], preferred_element_type=f32) + tb3_r[...]


def _tc_mlp(dense, embz, W1, b1, W2, b2, T1, tb1, T2, tb2, T3, tb3):
    in_top = _EMB_DIM * (_NUM_TABLES + 1)
    const = lambda i: (0, 0)
    return pl.pallas_call(
        _mlp_body,
        grid=(_B // _BT,),
        in_specs=[
            pl.BlockSpec((_BT, _NUM_DENSE), lambda i: (i, 0)),
            pl.BlockSpec((_BT, _NUM_TABLES * _EMB_DIM), lambda i: (i, 0)),
            pl.BlockSpec((_NUM_DENSE, _EMB_DIM), const),
            pl.BlockSpec((1, _EMB_DIM), const),
            pl.BlockSpec((_EMB_DIM, _EMB_DIM), const),
            pl.BlockSpec((1, _EMB_DIM), const),
            pl.BlockSpec((in_top, 512), const),
            pl.BlockSpec((1, 512), const),
            pl.BlockSpec((512, 256), const),
            pl.BlockSpec((1, 256), const),
            pl.BlockSpec((256, 1), const),
            pl.BlockSpec((1, 1), const),
        ],
        out_specs=pl.BlockSpec((_BT, 1), lambda i: (i, 0)),
        out_shape=jax.ShapeDtypeStruct((_B, 1), jnp.float32),
    )(dense, embz, W1, b1.reshape(1, -1), W2, b2.reshape(1, -1), T1,
      tb1.reshape(1, -1), T2, tb2.reshape(1, -1), T3, tb3.reshape(1, -1))


def kernel(dense, emb_idx, tables, W1, b1, W2, b2, T1, tb1, T2, tb2, T3, tb3):
    tables_flat = tables.reshape(_NUM_TABLES * _VOCAB, _EMB_DIM)
    idx_flat = emb_idx.astype(jnp.int32).reshape(-1)
    embz = _sc_gather(tables_flat, idx_flat)
    return _tc_mlp(dense, embz, W1, b1, W2, b2, T1, tb1, T2, tb2, T3, tb3)


# speedup vs baseline: 26.2602x; 1.0566x over previous
import functools

import jax
import jax.numpy as jnp
from jax import lax
from jax.experimental import pallas as pl
from jax.experimental.pallas import tpu as pltpu
from jax.experimental.pallas import tpu_sc as plsc

_EMB_DIM = 128
_NUM_DENSE = 13
_NUM_TABLES = 26
_VOCAB = 10000
_B = 16384
_SPLIT = 4

_NC = 2
_NS = 16
_NW = _NC * _NS

_CH = 128


def _make_sc_gather(nb):
    n_rows = _NUM_TABLES * nb
    ch_per_w = n_rows // (_NW * _CH)
    ch_per_table = nb // _CH
    assert n_rows % (_NW * _CH) == 0 and ch_per_w % 2 == 0
    mesh = plsc.VectorSubcoreMesh(core_axis_name="c", subcore_axis_name="s")

    @functools.partial(
        pl.kernel,
        mesh=mesh,
        out_type=jax.ShapeDtypeStruct((nb, _NUM_TABLES * _EMB_DIM),
                                      jnp.float32),
        scratch_types=[
            pltpu.VMEM((2, _CH), jnp.int32),
            pltpu.VMEM((2, _CH, _EMB_DIM), jnp.float32),
            pltpu.SemaphoreType.DMA,
            pltpu.SemaphoreType.DMA,
        ],
    )
    def gather(tab_hbm, idx_hbm, out_hbm, idx_v, rows_v, gsem0, gsem1):
        wid = lax.axis_index("s") * _NC + lax.axis_index("c")
        g0 = wid * ch_per_w
        gsems = (gsem0, gsem1)

        def load_idx(g, buf):
            pltpu.sync_copy(idx_hbm.at[pl.ds(g * _CH, _CH)], idx_v.at[buf])
            off = (g // ch_per_table) * _VOCAB
            for j in range(_CH // 16):
                sl = pl.ds(j * 16, 16)
                idx_v[buf, sl] = idx_v[buf, sl] + off

        def fire(g, buf):
            pltpu.async_copy(tab_hbm.at[idx_v.at[buf]], rows_v.at[buf],
                             gsems[buf])

        def wait_and_store(g, buf):
            pltpu.make_async_copy(tab_hbm.at[idx_v.at[buf]], rows_v.at[buf],
                                  gsems[buf]).wait()
            t = g // ch_per_table
            c = g - t * ch_per_table
            pltpu.sync_copy(
                rows_v.at[buf],
                out_hbm.at[pl.ds(c * _CH, _CH), pl.ds(t * _EMB_DIM, _EMB_DIM)])

        load_idx(g0, 0)
        fire(g0, 0)

        def body(p, carry):
            g = g0 + 2 * p
            load_idx(g + 1, 1)
            fire(g + 1, 1)
            wait_and_store(g, 0)

            @pl.when(p < ch_per_w // 2 - 1)
            def _():
                load_idx(g + 2, 0)
                fire(g + 2, 0)

            wait_and_store(g + 1, 1)
            return carry

        lax.fori_loop(0, ch_per_w // 2, body, 0)

    return gather


_BT = 512


def _mlp_body(dense_r, emb_r, W1_r, b1_r, W2_r, b2_r, T1_r, tb1_r, T2_r,
              tb2_r, T3_r, tb3_r, out_r):
    f32 = jnp.float32
    bf16 = jnp.bfloat16
    h = jnp.maximum(
        jnp.dot(dense_r[...], W1_r[...], preferred_element_type=f32) + b1_r[...],
        0.0)
    bot = jnp.dot(h, W2_r[...], preferred_element_type=f32) + b2_r[...]
    acc = jnp.dot(bot.astype(bf16), T1_r[0:_EMB_DIM, :].astype(bf16),
                  preferred_element_type=f32)
    acc = acc + jnp.dot(emb_r[...].astype(bf16),
                        T1_r[_EMB_DIM:, :].astype(bf16),
                        preferred_element_type=f32)
    h1 = jnp.maximum(acc + tb1_r[...], 0.0)
    h2 = jnp.maximum(
        jnp.dot(h1, T2_r[...], preferred_element_type=f32) + tb2_r[...], 0.0)
    out_r[...] = jnp.dot(h2, T3_r[...], preferred_element_type=f32) + tb3_r[...]


def _tc_mlp(nb, dense, embz, W1, b1, W2, b2, T1, tb1, T2, tb2, T3, tb3):
    in_top = _EMB_DIM * (_NUM_TABLES + 1)
    const = lambda i: (0, 0)
    return pl.pallas_call(
        _mlp_body,
        grid=(nb // _BT,),
        in_specs=[
            pl.BlockSpec((_BT, _NUM_DENSE), lambda i: (i, 0)),
            pl.BlockSpec((_BT, _NUM_TABLES * _EMB_DIM), lambda i: (i, 0)),
            pl.BlockSpec((_NUM_DENSE, _EMB_DIM), const),
            pl.BlockSpec((1, _EMB_DIM), const),
            pl.BlockSpec((_EMB_DIM, _EMB_DIM), const),
            pl.BlockSpec((1, _EMB_DIM), const),
            pl.BlockSpec((in_top, 512), const),
            pl.BlockSpec((1, 512), const),
            pl.BlockSpec((512, 256), const),
            pl.BlockSpec((1, 256), const),
            pl.BlockSpec((256, 1), const),
            pl.BlockSpec((1, 1), const),
        ],
        out_specs=pl.BlockSpec((_BT, 1), lambda i: (i, 0)),
        out_shape=jax.ShapeDtypeStruct((nb, 1), jnp.float32),
    )(dense, embz, W1, b1.reshape(1, -1), W2, b2.reshape(1, -1), T1,
      tb1.reshape(1, -1), T2, tb2.reshape(1, -1), T3, tb3.reshape(1, -1))


def kernel(dense, emb_idx, tables, W1, b1, W2, b2, T1, tb1, T2, tb2, T3, tb3):
    tables_flat = tables.reshape(_NUM_TABLES * _VOCAB, _EMB_DIM)
    idx32 = emb_idx.astype(jnp.int32)
    nb = _B // _SPLIT
    sc_gather = _make_sc_gather(nb)
    outs = []
    for s in range(_SPLIT):
        sl = slice(s * nb, (s + 1) * nb)
        embz = sc_gather(tables_flat, idx32[:, sl].reshape(-1))
        outs.append(_tc_mlp(nb, dense[sl], embz, W1, b1, W2, b2, T1, tb1, T2,
                            tb2, T3, tb3))
    return jnp.concatenate(outs, axis=0)


# speedup vs baseline: 26.4615x; 1.0077x over previous
import functools

import jax
import jax.numpy as jnp
from jax import lax
from jax.experimental import pallas as pl
from jax.experimental.pallas import tpu as pltpu
from jax.experimental.pallas import tpu_sc as plsc

_EMB_DIM = 128
_NUM_DENSE = 13
_NUM_TABLES = 26
_VOCAB = 10000
_B = 16384
_SPLIT = 4

_NC = 2
_NS = 16
_NW = _NC * _NS

_CH = 64
_NBUF = 4


def _make_sc_gather(nb):
    n_rows = _NUM_TABLES * nb
    ch_per_w = n_rows // (_NW * _CH)
    ch_per_table = nb // _CH
    n_quads = ch_per_w // _NBUF
    assert n_rows % (_NW * _CH) == 0 and ch_per_w % _NBUF == 0
    mesh = plsc.VectorSubcoreMesh(core_axis_name="c", subcore_axis_name="s")

    @functools.partial(
        pl.kernel,
        mesh=mesh,
        out_type=jax.ShapeDtypeStruct((nb, _NUM_TABLES * _EMB_DIM),
                                      jnp.float32),
        scratch_types=[
            pltpu.VMEM((ch_per_w, _CH), jnp.int32),
            pltpu.VMEM((_NBUF, _CH, _EMB_DIM), jnp.float32),
        ] + [pltpu.SemaphoreType.DMA] * (2 * _NBUF),
    )
    def gather(tab_hbm, idx_hbm, out_hbm, idx_all, rows_v, *sems):
        gsems, wsems = sems[:_NBUF], sems[_NBUF:]
        wid = lax.axis_index("s") * _NC + lax.axis_index("c")
        g0 = wid * ch_per_w

        pltpu.sync_copy(idx_hbm.at[wid], idx_all)
        for k in range(ch_per_w):
            off = ((g0 + k) // ch_per_table) * _VOCAB
            for j in range(_CH // 16):
                sl = pl.ds(j * 16, 16)
                idx_all[k, sl] = idx_all[k, sl] + off

        def fire_gather(k, b):
            pltpu.async_copy(tab_hbm.at[idx_all.at[k]], rows_v.at[b],
                             gsems[b])

        def wait_gather(b):
            pltpu.make_async_copy(tab_hbm.at[idx_all.at[0]], rows_v.at[b],
                                  gsems[b]).wait()

        def out_slab(k):
            g = g0 + k
            t = g // ch_per_table
            c = g - t * ch_per_table
            return out_hbm.at[pl.ds(c * _CH, _CH),
                              pl.ds(t * _EMB_DIM, _EMB_DIM)]

        def fire_write(k, b):
            pltpu.async_copy(rows_v.at[b], out_slab(k), wsems[b])

        def wait_write(b):
            pltpu.make_async_copy(rows_v.at[b], out_slab(0), wsems[b]).wait()

        for b in range(_NBUF):
            fire_gather(b, b)

        def body(p, carry):
            k = _NBUF * p
            for b in range(_NBUF):
                wait_gather(b)
                fire_write(k + b, b)

            @pl.when(p < n_quads - 1)
            def _():
                for b in range(_NBUF):
                    wait_write(b)
                    fire_gather(k + _NBUF + b, b)

            return carry

        lax.fori_loop(0, n_quads, body, 0)
        for b in range(_NBUF):
            wait_write(b)

    return gather


_BT = 512


def _mlp_body(dense_r, emb_r, W1_r, b1_r, W2_r, b2_r, T1_r, tb1_r, T2_r,
              tb2_r, T3_r, tb3_r, out_r):
    f32 = jnp.float32
    bf16 = jnp.bfloat16
    h = jnp.maximum(
        jnp.dot(dense_r[...], W1_r[...], preferred_element_type=f32) + b1_r[...],
        0.0)
    bot = jnp.dot(h, W2_r[...], preferred_element_type=f32) + b2_r[...]
    acc = jnp.dot(bot.astype(bf16), T1_r[0:_EMB_DIM, :].astype(bf16),
                  preferred_element_type=f32)
    acc = acc + jnp.dot(emb_r[...].astype(bf16),
                        T1_r[_EMB_DIM:, :].astype(bf16),
                        preferred_element_type=f32)
    h1 = jnp.maximum(acc + tb1_r[...], 0.0)
    h2 = jnp.maximum(
        jnp.dot(h1, T2_r[...], preferred_element_type=f32) + tb2_r[...], 0.0)
    out_r[...] = jnp.dot(h2, T3_r[...], preferred_element_type=f32) + tb3_r[...]


def _tc_mlp(nb, dense, embz, W1, b1, W2, b2, T1, tb1, T2, tb2, T3, tb3):
    in_top = _EMB_DIM * (_NUM_TABLES + 1)
    const = lambda i: (0, 0)
    return pl.pallas_call(
        _mlp_body,
        grid=(nb // _BT,),
        in_specs=[
            pl.BlockSpec((_BT, _NUM_DENSE), lambda i: (i, 0)),
            pl.BlockSpec((_BT, _NUM_TABLES * _EMB_DIM), lambda i: (i, 0)),
            pl.BlockSpec((_NUM_DENSE, _EMB_DIM), const),
            pl.BlockSpec((1, _EMB_DIM), const),
            pl.BlockSpec((_EMB_DIM, _EMB_DIM), const),
            pl.BlockSpec((1, _EMB_DIM), const),
            pl.BlockSpec((in_top, 512), const),
            pl.BlockSpec((1, 512), const),
            pl.BlockSpec((512, 256), const),
            pl.BlockSpec((1, 256), const),
            pl.BlockSpec((256, 1), const),
            pl.BlockSpec((1, 1), const),
        ],
        out_specs=pl.BlockSpec((_BT, 1), lambda i: (i, 0)),
        out_shape=jax.ShapeDtypeStruct((nb, 1), jnp.float32),
    )(dense, embz, W1, b1.reshape(1, -1), W2, b2.reshape(1, -1), T1,
      tb1.reshape(1, -1), T2, tb2.reshape(1, -1), T3, tb3.reshape(1, -1))


def kernel(dense, emb_idx, tables, W1, b1, W2, b2, T1, tb1, T2, tb2, T3, tb3):
    tables_flat = tables.reshape(_NUM_TABLES * _VOCAB, _EMB_DIM)
    idx32 = emb_idx.astype(jnp.int32)
    nb = _B // _SPLIT
    sc_gather = _make_sc_gather(nb)
    outs = []
    for s in range(_SPLIT):
        sl = slice(s * nb, (s + 1) * nb)
        nch = _NUM_TABLES * nb // (_NW * _CH)
        embz = sc_gather(tables_flat, idx32[:, sl].reshape(_NW, nch, _CH))
        outs.append(_tc_mlp(nb, dense[sl], embz, W1, b1, W2, b2, T1, tb1, T2,
                            tb2, T3, tb3))
    return jnp.concatenate(outs, axis=0)
